# Initial kernel scaffold; baseline (speedup 1.0000x reference)
import jax
import jax.numpy as jnp
from jax.experimental import pallas as pl


def kernel(rgb, xyzin, xyzout, W1, b1, g1, be1, W2, b2, g2, be2, W3, b3, g3, be3):
    raise NotImplementedError("write your pallas kernel here")


# speedup vs baseline: 23.1283x; 23.1283x over previous
import functools

import jax
import jax.numpy as jnp
from jax import lax
from jax.experimental import pallas as pl
from jax.experimental.pallas import tpu as pltpu
from jax.experimental.pallas import tpu_sc as plsc


def _mlp3_body(x_ref, w1_ref, w2_ref, w3_ref, b1_ref, g1_ref, be1_ref,
               b2_ref, g2_ref, be2_ref, b3_ref, g3_ref, be3_ref, out_ref):
    def layer(x, wt, b, g, be):
        y = lax.dot_general(x, wt, (((1,), (0,)), ((), ())),
                            preferred_element_type=jnp.float32)
        y = y + b
        mean = jnp.mean(y, axis=0, keepdims=True)
        var = jnp.mean(jnp.square(y - mean), axis=0, keepdims=True)
        y = (y - mean) / jnp.sqrt(var + 1e-5)
        y = y * g + be
        return jnp.maximum(y, 0.0)

    y = layer(x_ref[...], w1_ref[...], b1_ref[...], g1_ref[...], be1_ref[...])
    y = layer(y, w2_ref[...], b2_ref[...], g2_ref[...], be2_ref[...])
    out_ref[...] = layer(y, w3_ref[...], b3_ref[...], g3_ref[...], be3_ref[...])


def _mlp3(xt, w1t, w2t, w3t, vecs, interpret=False):
    n, _ = xt.shape
    c = w1t.shape[1]
    return pl.pallas_call(
        _mlp3_body,
        out_shape=jax.ShapeDtypeStruct((n, c), jnp.float32),
        interpret=interpret,
    )(xt, w1t, w2t, w3t, *vecs)


def _top3_body(nin, xyzout_ref, xyzint_ref, widx_ref, wrep_ref):
    b = pl.program_id(0)
    q = xyzout_ref[0]
    pt = xyzint_ref[0]
    bm = q.shape[0]
    qp = lax.dot_general(q, pt, (((1,), (0,)), ((), ())),
                         preferred_element_type=jnp.float32)
    qq = jnp.sum(q * q, axis=1, keepdims=True)
    pp = jnp.sum(pt * pt, axis=0, keepdims=True)
    d2 = qq + pp - 2.0 * qp
    d2 = jnp.where(d2 < 0, jnp.float32(1e-7), d2)
    d2 = jnp.square(jnp.sqrt(d2))

    iota = lax.broadcasted_iota(jnp.int32, (bm, nin), 1)
    cur = d2
    vals, inds = [], []
    for _ in range(3):
        mk = jnp.min(cur, axis=1, keepdims=True)
        ik = jnp.min(jnp.where(cur == mk, iota, nin), axis=1, keepdims=True)
        cur = jnp.where(iota == ik, jnp.float32(jnp.inf), cur)
        vals.append(mk)
        inds.append(ik)

    w0, w1, w2 = (1.0 / v for v in vals)
    ws = (w0 + w1) + w2
    wn = [w0 / ws, w1 / ws, w2 / ws]

    widx_ref[0] = jnp.concatenate([b * nin + i for i in inds], axis=1)
    wrep_ref[0] = jnp.concatenate(
        [jnp.broadcast_to(w, (bm, 16)) for w in wn], axis=1)


def _top3(xyzout, xyzint, bm=512, interpret=False):
    bb, m, _ = xyzout.shape
    nin = xyzint.shape[2]
    grid = (bb, m // bm)
    return pl.pallas_call(
        functools.partial(_top3_body, nin),
        grid=grid,
        in_specs=[
            pl.BlockSpec((1, bm, 3), lambda b, i: (b, i, 0)),
            pl.BlockSpec((1, 3, nin), lambda b, i: (b, 0, 0)),
        ],
        out_specs=[
            pl.BlockSpec((1, bm, 3), lambda b, i: (b, i, 0)),
            pl.BlockSpec((1, bm, 48), lambda b, i: (b, i, 0)),
        ],
        out_shape=[
            jax.ShapeDtypeStruct((bb, m, 3), jnp.int32),
            jax.ShapeDtypeStruct((bb, m, 48), jnp.float32),
        ],
        interpret=interpret,
    )(xyzout, xyzint)


_NC, _NS = 2, 16
_QC = 32


@functools.cache
def _sc_interp_builder(bq, c):
    nw = _NC * _NS
    q_per_w = bq // nw
    n_chunks = q_per_w // _QC
    mesh = plsc.VectorSubcoreMesh(core_axis_name="c", subcore_axis_name="s",
                                  num_cores=_NC, num_subcores=_NS)

    def body(table_hbm, fidx_hbm, wrep_hbm, out_hbm,
             idx_v, rows_v, w_v, out_v, sem):
        wid = lax.axis_index("s") * _NC + lax.axis_index("c")
        for ch in range(n_chunks):
            base = wid * q_per_w + ch * _QC
            pltpu.sync_copy(fidx_hbm.at[pl.ds(base * 3, _QC * 3)], idx_v)
            pltpu.async_copy(table_hbm.at[idx_v], rows_v, sem).wait()
            pltpu.sync_copy(wrep_hbm.at[pl.ds(base, _QC)], w_v)

            def qbody(qi, _):
                wv0 = w_v[qi, pl.ds(0, 16)]
                wv1 = w_v[qi, pl.ds(16, 16)]
                wv2 = w_v[qi, pl.ds(32, 16)]
                for cb in range(c // 16):
                    sl = pl.ds(cb * 16, 16)
                    acc = (rows_v[3 * qi, sl] * wv0
                           + rows_v[3 * qi + 1, sl] * wv1
                           + rows_v[3 * qi + 2, sl] * wv2)
                    out_v[qi, sl] = acc
                return 0

            lax.fori_loop(0, _QC, qbody, 0)
            pltpu.sync_copy(out_v, out_hbm.at[pl.ds(base, _QC)])

    return pl.kernel(
        body,
        out_type=jax.ShapeDtypeStruct((bq, c), jnp.float32),
        mesh=mesh,
        scratch_types=[
            pltpu.VMEM((_QC * 3,), jnp.int32),
            pltpu.VMEM((_QC * 3, c), jnp.float32),
            pltpu.VMEM((_QC, 48), jnp.float32),
            pltpu.VMEM((_QC, c), jnp.float32),
            pltpu.SemaphoreType.DMA,
        ],
    )


def kernel(rgb, xyzin, xyzout, W1, b1, g1, be1, W2, b2, g2, be2, W3, b3, g3, be3):
    bb, cin, nin = rgb.shape
    m = xyzout.shape[1]
    c = W1.shape[0]

    xt = jnp.transpose(rgb, (0, 2, 1)).reshape(bb * nin, cin)
    vecs = [v.reshape(1, c) for v in (b1, g1, be1, b2, g2, be2, b3, g3, be3)]
    table = _mlp3(xt, W1.T, W2.T, W3.T, vecs)

    xyzint = jnp.transpose(xyzin, (0, 2, 1))
    widx, wrep = _top3(xyzout, xyzint)
    fidx = widx.reshape(bb * m * 3)
    wrep = wrep.reshape(bb * m, 48)

    out = _sc_interp_builder(bb * m, c)(table, fidx, wrep)
    return jnp.transpose(out.reshape(bb, m, c), (0, 2, 1))


# speedup vs baseline: 30.4099x; 1.3148x over previous
import functools

import jax
import jax.numpy as jnp
from jax import lax
from jax.experimental import pallas as pl
from jax.experimental.pallas import tpu as pltpu
from jax.experimental.pallas import tpu_sc as plsc


def _mlp3_body(x_ref, w1_ref, w2_ref, w3_ref, b1_ref, g1_ref, be1_ref,
               b2_ref, g2_ref, be2_ref, b3_ref, g3_ref, be3_ref, out_ref):
    def layer(x, wt, b, g, be):
        y = lax.dot_general(x, wt, (((1,), (0,)), ((), ())),
                            preferred_element_type=jnp.float32)
        y = y + b
        mean = jnp.mean(y, axis=0, keepdims=True)
        var = jnp.mean(jnp.square(y - mean), axis=0, keepdims=True)
        y = (y - mean) / jnp.sqrt(var + 1e-5)
        y = y * g + be
        return jnp.maximum(y, 0.0)

    y = layer(x_ref[...], w1_ref[...], b1_ref[...], g1_ref[...], be1_ref[...])
    y = layer(y, w2_ref[...], b2_ref[...], g2_ref[...], be2_ref[...])
    out_ref[...] = layer(y, w3_ref[...], b3_ref[...], g3_ref[...], be3_ref[...])


def _mlp3(xt, w1t, w2t, w3t, vecs, interpret=False):
    n, _ = xt.shape
    c = w1t.shape[1]
    return pl.pallas_call(
        _mlp3_body,
        out_shape=jax.ShapeDtypeStruct((n, c), jnp.float32),
        interpret=interpret,
    )(xt, w1t, w2t, w3t, *vecs)


def _top3_body(nin, xyzout_ref, xyzint_ref, widx_ref, wrep_ref):
    b = pl.program_id(0)
    q = xyzout_ref[0]
    pt = xyzint_ref[0]
    bm = q.shape[0]
    qq = jnp.sum(q * q, axis=1, keepdims=True)
    pp = jnp.sum(pt * pt, axis=0, keepdims=True)
    qa = jnp.concatenate([-2.0 * q, qq, jnp.ones_like(qq)], axis=1)
    pa = jnp.concatenate([pt, jnp.ones_like(pp), pp], axis=0)
    d2 = lax.dot_general(qa, pa, (((1,), (0,)), ((), ())),
                         preferred_element_type=jnp.float32)
    d2 = jnp.where(d2 < 0, jnp.float32(1e-7), d2)

    iota = lax.broadcasted_iota(jnp.int32, (bm, nin), 1)
    cur = d2
    vals, inds = [], []
    for _ in range(3):
        mk = jnp.min(cur, axis=1, keepdims=True)
        ik = jnp.min(jnp.where(cur == mk, iota, nin), axis=1, keepdims=True)
        cur = jnp.where(iota == ik, jnp.float32(jnp.inf), cur)
        vals.append(mk)
        inds.append(ik)

    w0, w1, w2 = (1.0 / v for v in vals)
    ws = (w0 + w1) + w2
    wn = [w0 / ws, w1 / ws, w2 / ws]

    widx_ref[0] = jnp.concatenate([b * nin + i for i in inds], axis=1)
    wrep_ref[0] = jnp.concatenate(
        [jnp.broadcast_to(w, (bm, 16)) for w in wn], axis=1)


def _top3(xyzout, xyzint, bm=512, interpret=False):
    bb, m, _ = xyzout.shape
    nin = xyzint.shape[2]
    grid = (bb, m // bm)
    return pl.pallas_call(
        functools.partial(_top3_body, nin),
        grid=grid,
        in_specs=[
            pl.BlockSpec((1, bm, 3), lambda b, i: (b, i, 0)),
            pl.BlockSpec((1, 3, nin), lambda b, i: (b, 0, 0)),
        ],
        out_specs=[
            pl.BlockSpec((1, bm, 3), lambda b, i: (b, i, 0)),
            pl.BlockSpec((1, bm, 48), lambda b, i: (b, i, 0)),
        ],
        out_shape=[
            jax.ShapeDtypeStruct((bb, m, 3), jnp.int32),
            jax.ShapeDtypeStruct((bb, m, 48), jnp.float32),
        ],
        interpret=interpret,
    )(xyzout, xyzint)


_NC, _NS = 2, 16
_QC = 32


@functools.cache
def _sc_interp_builder(bq, c):
    nw = _NC * _NS
    q_per_w = bq // nw
    n_chunks = q_per_w // _QC
    mesh = plsc.VectorSubcoreMesh(core_axis_name="c", subcore_axis_name="s",
                                  num_cores=_NC, num_subcores=_NS)

    def body(table_hbm, fidx_hbm, wrep_hbm, out_hbm,
             idx_all, w0, w1, rows0, rows1, out0, out1,
             gsem0, gsem1, wsem0, wsem1, osem0, osem1):
        wid = lax.axis_index("s") * _NC + lax.axis_index("c")
        pltpu.sync_copy(fidx_hbm.at[pl.ds(wid * n_chunks, n_chunks)], idx_all)

        rows = (rows0, rows1)
        ws = (w0, w1)
        outs = (out0, out1)
        gsems = (gsem0, gsem1)
        wsems = (wsem0, wsem1)
        osems = (osem0, osem1)
        gathers = [None, None]
        wcopies = [None, None]
        stores = [None, None]

        def fetch(ch, slot):
            gathers[slot] = pltpu.async_copy(
                table_hbm.at[idx_all.at[ch]], rows[slot], gsems[slot])
            wcopies[slot] = pltpu.async_copy(
                wrep_hbm.at[pl.ds(wid * q_per_w + ch * _QC, _QC)],
                ws[slot], wsems[slot])

        fetch(0, 0)
        for ch in range(n_chunks):
            cur, nxt = ch % 2, (ch + 1) % 2
            if ch + 1 < n_chunks:
                fetch(ch + 1, nxt)
            gathers[cur].wait()
            wcopies[cur].wait()
            if stores[cur] is not None:
                stores[cur].wait()
            rows_v, w_v, out_v = rows[cur], ws[cur], outs[cur]

            def qbody(qi, _):
                wv0 = w_v[qi, pl.ds(0, 16)]
                wv1 = w_v[qi, pl.ds(16, 16)]
                wv2 = w_v[qi, pl.ds(32, 16)]
                for cb in range(c // 16):
                    sl = pl.ds(cb * 16, 16)
                    acc = (rows_v[3 * qi, sl] * wv0
                           + rows_v[3 * qi + 1, sl] * wv1
                           + rows_v[3 * qi + 2, sl] * wv2)
                    out_v[qi, sl] = acc
                return 0

            lax.fori_loop(0, _QC, qbody, 0)
            stores[cur] = pltpu.async_copy(
                out_v, out_hbm.at[pl.ds(wid * q_per_w + ch * _QC, _QC)],
                osems[cur])
        for st in stores:
            if st is not None:
                st.wait()

    return pl.kernel(
        body,
        out_type=jax.ShapeDtypeStruct((bq, c), jnp.float32),
        mesh=mesh,
        scratch_types=[
            pltpu.VMEM((n_chunks, _QC * 3), jnp.int32),
            pltpu.VMEM((_QC, 48), jnp.float32),
            pltpu.VMEM((_QC, 48), jnp.float32),
            pltpu.VMEM((_QC * 3, c), jnp.float32),
            pltpu.VMEM((_QC * 3, c), jnp.float32),
            pltpu.VMEM((_QC, c), jnp.float32),
            pltpu.VMEM((_QC, c), jnp.float32),
            pltpu.SemaphoreType.DMA,
            pltpu.SemaphoreType.DMA,
            pltpu.SemaphoreType.DMA,
            pltpu.SemaphoreType.DMA,
            pltpu.SemaphoreType.DMA,
            pltpu.SemaphoreType.DMA,
        ],
    )


def kernel(rgb, xyzin, xyzout, W1, b1, g1, be1, W2, b2, g2, be2, W3, b3, g3, be3):
    bb, cin, nin = rgb.shape
    m = xyzout.shape[1]
    c = W1.shape[0]

    xt = jnp.transpose(rgb, (0, 2, 1)).reshape(bb * nin, cin)
    vecs = [v.reshape(1, c) for v in (b1, g1, be1, b2, g2, be2, b3, g3, be3)]
    table = _mlp3(xt, W1.T, W2.T, W3.T, vecs)

    xyzint = jnp.transpose(xyzin, (0, 2, 1))
    widx, wrep = _top3(xyzout, xyzint)
    fidx = widx.reshape(bb * m * 3 // (_QC * 3), _QC * 3)
    wrep = wrep.reshape(bb * m, 48)

    out = _sc_interp_builder(bb * m, c)(table, fidx, wrep)
    return jnp.transpose(out.reshape(bb, m, c), (0, 2, 1))


# speedup vs baseline: 38.1288x; 1.2538x over previous
import functools

import jax
import jax.numpy as jnp
from jax import lax
from jax.experimental import pallas as pl
from jax.experimental.pallas import tpu as pltpu
from jax.experimental.pallas import tpu_sc as plsc


def _mlp3_body(x_ref, w1_ref, w2_ref, w3_ref, b1_ref, g1_ref, be1_ref,
               b2_ref, g2_ref, be2_ref, b3_ref, g3_ref, be3_ref, out_ref):
    def layer(x, wt, b, g, be):
        y = lax.dot_general(x, wt, (((1,), (0,)), ((), ())),
                            preferred_element_type=jnp.float32)
        y = y + b
        mean = jnp.mean(y, axis=0, keepdims=True)
        var = jnp.mean(jnp.square(y - mean), axis=0, keepdims=True)
        y = (y - mean) / jnp.sqrt(var + 1e-5)
        y = y * g + be
        return jnp.maximum(y, 0.0)

    y = layer(x_ref[...], w1_ref[...], b1_ref[...], g1_ref[...], be1_ref[...])
    y = layer(y, w2_ref[...], b2_ref[...], g2_ref[...], be2_ref[...])
    out_ref[...] = layer(y, w3_ref[...], b3_ref[...], g3_ref[...], be3_ref[...])


def _mlp3(xt, w1t, w2t, w3t, vecs, interpret=False):
    n, _ = xt.shape
    c = w1t.shape[1]
    return pl.pallas_call(
        _mlp3_body,
        out_shape=jax.ShapeDtypeStruct((n, c), jnp.float32),
        interpret=interpret,
    )(xt, w1t, w2t, w3t, *vecs)


def _top3_body(nin, xyzout_ref, xyzint_ref, widx_ref, wrep_ref):
    b = pl.program_id(0)
    q = xyzout_ref[0]
    pt = xyzint_ref[0]
    bm = q.shape[0]
    qq = jnp.sum(q * q, axis=1, keepdims=True)
    pp = jnp.sum(pt * pt, axis=0, keepdims=True)
    qa = jnp.concatenate([-2.0 * q, qq, jnp.ones_like(qq)], axis=1)
    pa = jnp.concatenate([pt, jnp.ones_like(pp), pp], axis=0)
    d2 = lax.dot_general(qa, pa, (((1,), (0,)), ((), ())),
                         preferred_element_type=jnp.float32)
    d2 = jnp.where(d2 < 0, jnp.float32(1e-7), d2)

    iota = lax.broadcasted_iota(jnp.int32, (bm, nin), 1).astype(jnp.float32)
    cur = d2
    vals, inds = [], []
    for k in range(3):
        mk = jnp.min(cur, axis=1, keepdims=True)
        ik = jnp.min(jnp.where(cur == mk, iota, jnp.float32(nin)),
                     axis=1, keepdims=True)
        if k < 2:
            cur = jnp.where(iota == ik, jnp.float32(jnp.inf), cur)
        vals.append(mk)
        inds.append(ik.astype(jnp.int32))

    w0, w1, w2 = (1.0 / v for v in vals)
    ws = (w0 + w1) + w2
    wn = [w0 / ws, w1 / ws, w2 / ws]

    widx_ref[0] = jnp.concatenate([b * nin + i for i in inds], axis=1)
    wrep_ref[0] = jnp.concatenate(
        [jnp.broadcast_to(w, (bm, 16)) for w in wn], axis=1)


def _top3(xyzout, xyzint, bm=1024, interpret=False):
    bb, m, _ = xyzout.shape
    nin = xyzint.shape[2]
    grid = (bb, m // bm)
    return pl.pallas_call(
        functools.partial(_top3_body, nin),
        grid=grid,
        in_specs=[
            pl.BlockSpec((1, bm, 3), lambda b, i: (b, i, 0)),
            pl.BlockSpec((1, 3, nin), lambda b, i: (b, 0, 0)),
        ],
        out_specs=[
            pl.BlockSpec((1, bm, 3), lambda b, i: (b, i, 0)),
            pl.BlockSpec((1, bm, 48), lambda b, i: (b, i, 0)),
        ],
        out_shape=[
            jax.ShapeDtypeStruct((bb, m, 3), jnp.int32),
            jax.ShapeDtypeStruct((bb, m, 48), jnp.float32),
        ],
        interpret=interpret,
    )(xyzout, xyzint)


_NC, _NS = 2, 16
_QC = 32


@functools.cache
def _sc_interp_builder(bq, c):
    nw = _NC * _NS
    q_per_w = bq // nw
    n_chunks = q_per_w // _QC
    mesh = plsc.VectorSubcoreMesh(core_axis_name="c", subcore_axis_name="s",
                                  num_cores=_NC, num_subcores=_NS)

    def body(table_hbm, fidx_hbm, wrep_hbm, out_hbm,
             idx_all, w0, w1, rows0, rows1, out0, out1,
             gsem0, gsem1, wsem0, wsem1, osem0, osem1):
        wid = lax.axis_index("s") * _NC + lax.axis_index("c")
        pltpu.sync_copy(fidx_hbm.at[pl.ds(wid * n_chunks, n_chunks)], idx_all)

        rows = (rows0, rows1)
        ws = (w0, w1)
        outs = (out0, out1)
        gsems = (gsem0, gsem1)
        wsems = (wsem0, wsem1)
        osems = (osem0, osem1)
        gathers = [None, None]
        wcopies = [None, None]
        stores = [None, None]

        def fetch(ch, slot):
            gathers[slot] = pltpu.async_copy(
                table_hbm.at[idx_all.at[ch]], rows[slot], gsems[slot])
            wcopies[slot] = pltpu.async_copy(
                wrep_hbm.at[pl.ds(wid * q_per_w + ch * _QC, _QC)],
                ws[slot], wsems[slot])

        fetch(0, 0)
        for ch in range(n_chunks):
            cur, nxt = ch % 2, (ch + 1) % 2
            if ch + 1 < n_chunks:
                fetch(ch + 1, nxt)
            gathers[cur].wait()
            wcopies[cur].wait()
            if stores[cur] is not None:
                stores[cur].wait()
            rows_v, w_v, out_v = rows[cur], ws[cur], outs[cur]

            @plsc.parallel_loop(0, _QC)
            def qbody(qi):
                wv0 = w_v[qi, pl.ds(0, 16)]
                wv1 = w_v[qi, pl.ds(16, 16)]
                wv2 = w_v[qi, pl.ds(32, 16)]
                for cb in range(c // 16):
                    sl = pl.ds(cb * 16, 16)
                    acc = (rows_v[3 * qi, sl] * wv0
                           + rows_v[3 * qi + 1, sl] * wv1
                           + rows_v[3 * qi + 2, sl] * wv2)
                    out_v[qi, sl] = acc
            stores[cur] = pltpu.async_copy(
                out_v, out_hbm.at[pl.ds(wid * q_per_w + ch * _QC, _QC)],
                osems[cur])
        for st in stores:
            if st is not None:
                st.wait()

    return pl.kernel(
        body,
        out_type=jax.ShapeDtypeStruct((bq, c), jnp.float32),
        mesh=mesh,
        scratch_types=[
            pltpu.VMEM((n_chunks, _QC * 3), jnp.int32),
            pltpu.VMEM((_QC, 48), jnp.float32),
            pltpu.VMEM((_QC, 48), jnp.float32),
            pltpu.VMEM((_QC * 3, c), jnp.float32),
            pltpu.VMEM((_QC * 3, c), jnp.float32),
            pltpu.VMEM((_QC, c), jnp.float32),
            pltpu.VMEM((_QC, c), jnp.float32),
            pltpu.SemaphoreType.DMA,
            pltpu.SemaphoreType.DMA,
            pltpu.SemaphoreType.DMA,
            pltpu.SemaphoreType.DMA,
            pltpu.SemaphoreType.DMA,
            pltpu.SemaphoreType.DMA,
        ],
    )


def kernel(rgb, xyzin, xyzout, W1, b1, g1, be1, W2, b2, g2, be2, W3, b3, g3, be3):
    bb, cin, nin = rgb.shape
    m = xyzout.shape[1]
    c = W1.shape[0]

    xt = jnp.transpose(rgb, (0, 2, 1)).reshape(bb * nin, cin)
    vecs = [v.reshape(1, c) for v in (b1, g1, be1, b2, g2, be2, b3, g3, be3)]
    table = _mlp3(xt, W1.T, W2.T, W3.T, vecs)

    xyzint = jnp.transpose(xyzin, (0, 2, 1))
    widx, wrep = _top3(xyzout, xyzint)
    fidx = widx.reshape(bb * m * 3 // (_QC * 3), _QC * 3)
    wrep = wrep.reshape(bb * m, 48)

    out = _sc_interp_builder(bb * m, c)(table, fidx, wrep)
    return jnp.transpose(out.reshape(bb, m, c), (0, 2, 1))
